# baseline (speedup 1.0000x reference)
import functools

import jax
import jax.numpy as jnp
from jax import lax
from jax.experimental import pallas as pl
from jax.experimental.pallas import tpu as pltpu
from jax.experimental.pallas import tpu_sc as plsc

N_WORKERS = 32
GROUP = 16
D = 64
DO = 65
DP = 128
EPS = 1e-8


def _rsqrt_newton(s):
    i = plsc.bitcast(s, jnp.int32)
    i = jnp.int32(0x5F3759DF) - (i >> 1)
    y = plsc.bitcast(i, jnp.float32)
    for _ in range(3):
        y = y * (1.5 - 0.5 * s * y * y)
    return y


def _sc_kernel(b, h):
    bpw = b // N_WORKERS
    chunk = h
    half = chunk // 2
    n_full = chunk // GROUP
    tail = chunk - n_full * GROUP
    mesh = plsc.VectorSubcoreMesh(core_axis_name="c", subcore_axis_name="s")

    @functools.partial(
        pl.kernel,
        mesh=mesh,
        out_type=jax.ShapeDtypeStruct((b, h, DP), jnp.float32),
        scratch_types=[
            pltpu.VMEM((bpw, 2, half), jnp.int32),
            pltpu.VMEM((2, chunk, DP), jnp.float32),
            pltpu.SemaphoreType.DMA,
            pltpu.SemaphoreType.DMA,
            pltpu.SemaphoreType.DMA,
        ],
        compiler_params=pltpu.CompilerParams(needs_layout_passes=False,
                                             use_tc_tiling_on_sc=False,
                                             skip_device_barrier=True),
    )
    def k(idx_hbm, table_hbm, out_hbm, idx_v, buf_v, gsem0, gsem1, osem):
        wid = lax.axis_index("s") * 2 + lax.axis_index("c")
        wbase = wid * bpw
        pltpu.sync_copy(idx_hbm.at[wid], idx_v)

        lane = lax.iota(jnp.int32, GROUP)

        def gather_copies(c, p, sem):
            return [
                pltpu.make_async_copy(
                    table_hbm.at[idx_v.at[c, 0]],
                    buf_v.at[p, pl.ds(0, half)], sem),
                pltpu.make_async_copy(
                    table_hbm.at[idx_v.at[c, 1]],
                    buf_v.at[p, pl.ds(half, half)], sem),
            ]

        def out_copy(c, p):
            return pltpu.make_async_copy(
                buf_v.at[p], out_hbm.at[wbase + c], osem)

        for g in gather_copies(0, 0, gsem0):
            g.start()

        def do_group(buf, base, nrows):
            parts = []
            for r in range(nrows):
                t0 = buf[base + r, pl.ds(0, 16)]
                t1 = buf[base + r, pl.ds(16, 16)]
                t2 = buf[base + r, pl.ds(32, 16)]
                t3 = buf[base + r, pl.ds(48, 16)]
                t = (t0 * t0 + t1 * t1) + (t2 * t2 + t3 * t3)
                parts.append(jnp.where(lane == r,
                                       lax.broadcast(jnp.sum(t), (GROUP,)),
                                       0.0))
            while len(parts) > 1:
                parts = [a + b for a, b in zip(parts[::2], parts[1::2])] + (
                    [parts[-1]] if len(parts) % 2 else [])
            s = parts[0]
            s = jnp.maximum(s, EPS)
            y = _rsqrt_newton(s)
            n = s * y
            en = jnp.exp(n)
            ien = 1.0 / en
            cosh = 0.5 * (en + ien)
            sf = 0.5 * (en - ien) * y
            for r in range(nrows):
                sfr = lax.broadcast(sf[r], (GROUP,))
                rr = jnp.full((GROUP,), base + r, jnp.int32)
                vs = [buf[base + r, pl.ds(16 * q, 16)] for q in range(4)]
                for q in range(4):
                    plsc.store_scatter(buf, [rr, 1 + 16 * q + lane],
                                       sfr * vs[q])
            plsc.store_scatter(buf, [base + lane,
                                     jnp.zeros((GROUP,), jnp.int32)], cosh,
                               mask=lane < nrows)

        def chunk_body(c, _):
            p = c % 2

            @pl.when(p == 0)
            def _w0():
                for g in gather_copies(c, 0, gsem0):
                    g.wait()

            @pl.when(p == 1)
            def _w1():
                for g in gather_copies(c, 1, gsem1):
                    g.wait()

            @pl.when(c >= 1)
            def _wo():
                out_copy(c - 1, 1 - p).wait()

            @pl.when(c + 1 < bpw)
            def _g():
                @pl.when(p == 0)
                def _g1():
                    for g in gather_copies(c + 1, 1, gsem1):
                        g.start()

                @pl.when(p == 1)
                def _g0():
                    for g in gather_copies(c + 1, 0, gsem0):
                        g.start()

            buf = buf_v.at[p]

            @plsc.parallel_loop(0, n_full, step=1, unroll=2)
            def _groups(g):
                do_group(buf, g * GROUP, GROUP)
            if tail:
                do_group(buf, n_full * GROUP, tail)
            out_copy(c, p).start()
            return _

        lax.fori_loop(0, bpw, chunk_body, 0)
        out_copy(bpw - 1, (bpw - 1) % 2).wait()

    return k


def kernel(x, embed_weight):
    b, h = x.shape
    idx = x.reshape(N_WORKERS, b // N_WORKERS, 2, h // 2).astype(jnp.int32)
    tbl = jnp.concatenate([embed_weight, embed_weight], axis=1)
    out = _sc_kernel(b, h)(idx, tbl)
    return lax.slice(out, (0, 0, 0), (b, h, DO))


# speedup vs baseline: 1.1216x; 1.1216x over previous
import functools

import jax
import jax.numpy as jnp
from jax import lax
from jax.experimental import pallas as pl
from jax.experimental.pallas import tpu as pltpu
from jax.experimental.pallas import tpu_sc as plsc

N_WORKERS = 32
GROUP = 16
D = 64
DO = 65
DP = 128
EPS = 1e-8


def _rsqrt_newton(s):
    i = plsc.bitcast(s, jnp.int32)
    i = jnp.int32(0x5F3759DF) - (i >> 1)
    y = plsc.bitcast(i, jnp.float32)
    for _ in range(3):
        y = y * (1.5 - 0.5 * s * y * y)
    return y


def _sc_kernel(b, h):
    bpw = b // N_WORKERS
    chunk = h
    half = chunk // 2
    n_full = chunk // GROUP
    tail = chunk - n_full * GROUP
    mesh = plsc.VectorSubcoreMesh(core_axis_name="c", subcore_axis_name="s")

    @functools.partial(
        pl.kernel,
        mesh=mesh,
        out_type=jax.ShapeDtypeStruct((b, h, DP), jnp.float32),
        scratch_types=[
            pltpu.VMEM((bpw, 2, half), jnp.int32),
            pltpu.VMEM((2, chunk, DP), jnp.float32),
            pltpu.SemaphoreType.DMA,
            pltpu.SemaphoreType.DMA,
            pltpu.SemaphoreType.DMA,
        ],
        compiler_params=pltpu.CompilerParams(needs_layout_passes=False,
                                             use_tc_tiling_on_sc=False,
                                             skip_device_barrier=True),
    )
    def k(idx_hbm, table_hbm, out_hbm, idx_v, buf_v, gsem0, gsem1, osem):
        wid = lax.axis_index("s") * 2 + lax.axis_index("c")
        wbase = wid * bpw
        pltpu.sync_copy(idx_hbm.at[wid], idx_v)

        lane = lax.iota(jnp.int32, GROUP)

        def gather_copies(c, p, sem):
            return [
                pltpu.make_async_copy(
                    table_hbm.at[idx_v.at[c, 0]],
                    buf_v.at[p, pl.ds(0, half)], sem),
                pltpu.make_async_copy(
                    table_hbm.at[idx_v.at[c, 1]],
                    buf_v.at[p, pl.ds(half, half)], sem),
            ]

        def out_copy(c, p):
            return pltpu.make_async_copy(
                buf_v.at[p, slice(None), pl.ds(0, 72)],
                out_hbm.at[wbase + c, slice(None), pl.ds(0, 72)], osem)

        for g in gather_copies(0, 0, gsem0):
            g.start()

        def do_group(buf, base, nrows):
            parts = []
            for r in range(nrows):
                t0 = buf[base + r, pl.ds(0, 16)]
                t1 = buf[base + r, pl.ds(16, 16)]
                t2 = buf[base + r, pl.ds(32, 16)]
                t3 = buf[base + r, pl.ds(48, 16)]
                t = (t0 * t0 + t1 * t1) + (t2 * t2 + t3 * t3)
                parts.append(jnp.where(lane == r,
                                       lax.broadcast(jnp.sum(t), (GROUP,)),
                                       0.0))
            while len(parts) > 1:
                parts = [a + b for a, b in zip(parts[::2], parts[1::2])] + (
                    [parts[-1]] if len(parts) % 2 else [])
            s = parts[0]
            s = jnp.maximum(s, EPS)
            y = _rsqrt_newton(s)
            n = s * y
            en = jnp.exp(n)
            ien = 1.0 / en
            cosh = 0.5 * (en + ien)
            sf = 0.5 * (en - ien) * y
            for r in range(nrows):
                sfr = lax.broadcast(sf[r], (GROUP,))
                rr = jnp.full((GROUP,), base + r, jnp.int32)
                vs = [buf[base + r, pl.ds(16 * q, 16)] for q in range(4)]
                for q in range(4):
                    plsc.store_scatter(buf, [rr, 1 + 16 * q + lane],
                                       sfr * vs[q])
            plsc.store_scatter(buf, [base + lane,
                                     jnp.zeros((GROUP,), jnp.int32)], cosh,
                               mask=lane < nrows)

        def chunk_body(c, _):
            p = c % 2

            @pl.when(p == 0)
            def _w0():
                for g in gather_copies(c, 0, gsem0):
                    g.wait()

            @pl.when(p == 1)
            def _w1():
                for g in gather_copies(c, 1, gsem1):
                    g.wait()

            @pl.when(c >= 1)
            def _wo():
                out_copy(c - 1, 1 - p).wait()

            @pl.when(c + 1 < bpw)
            def _g():
                @pl.when(p == 0)
                def _g1():
                    for g in gather_copies(c + 1, 1, gsem1):
                        g.start()

                @pl.when(p == 1)
                def _g0():
                    for g in gather_copies(c + 1, 0, gsem0):
                        g.start()

            buf = buf_v.at[p]

            @plsc.parallel_loop(0, n_full, step=1, unroll=2)
            def _groups(g):
                do_group(buf, g * GROUP, GROUP)
            if tail:
                do_group(buf, n_full * GROUP, tail)
            out_copy(c, p).start()
            return _

        lax.fori_loop(0, bpw, chunk_body, 0)
        out_copy(bpw - 1, (bpw - 1) % 2).wait()

    return k


def kernel(x, embed_weight):
    b, h = x.shape
    idx = x.reshape(N_WORKERS, b // N_WORKERS, 2, h // 2).astype(jnp.int32)
    tbl = jnp.pad(embed_weight, ((0, 0), (0, DP - D)))
    out = _sc_kernel(b, h)(idx, tbl)
    return lax.slice(out, (0, 0, 0), (b, h, DO))


# speedup vs baseline: 1.1367x; 1.0134x over previous
import functools

import jax
import jax.numpy as jnp
from jax import lax
from jax.experimental import pallas as pl
from jax.experimental.pallas import tpu as pltpu
from jax.experimental.pallas import tpu_sc as plsc

N_WORKERS = 32
GROUP = 16
D = 64
DO = 65
DP = 128
EPS = 1e-8


def _rsqrt_newton(s):
    i = plsc.bitcast(s, jnp.int32)
    i = jnp.int32(0x5F3759DF) - (i >> 1)
    y = plsc.bitcast(i, jnp.float32)
    for _ in range(3):
        y = y * (1.5 - 0.5 * s * y * y)
    return y


def _sc_kernel(b, h):
    bpw = b // N_WORKERS
    chunk = h
    half = chunk // 2
    n_full = chunk // GROUP
    tail = chunk - n_full * GROUP
    mesh = plsc.VectorSubcoreMesh(core_axis_name="c", subcore_axis_name="s")

    @functools.partial(
        pl.kernel,
        mesh=mesh,
        out_type=jax.ShapeDtypeStruct((b, h, DP), jnp.float32),
        scratch_types=[
            pltpu.VMEM((bpw, 2, half), jnp.int32),
            pltpu.VMEM((2, chunk, DP), jnp.float32),
            pltpu.SemaphoreType.DMA,
            pltpu.SemaphoreType.DMA,
            pltpu.SemaphoreType.DMA,
        ],
        compiler_params=pltpu.CompilerParams(needs_layout_passes=False,
                                             use_tc_tiling_on_sc=False,
                                             skip_device_barrier=True),
    )
    def k(idx_hbm, table_hbm, out_hbm, idx_v, buf_v, gsem0, gsem1, osem):
        wid = lax.axis_index("s") * 2 + lax.axis_index("c")
        wbase = wid * bpw
        pltpu.sync_copy(idx_hbm.at[wid], idx_v)

        lane = lax.iota(jnp.int32, GROUP)

        def gather_copies(c, p, sem):
            return [
                pltpu.make_async_copy(
                    table_hbm.at[idx_v.at[c, 0]],
                    buf_v.at[p, pl.ds(0, half)], sem),
                pltpu.make_async_copy(
                    table_hbm.at[idx_v.at[c, 1]],
                    buf_v.at[p, pl.ds(half, half)], sem),
            ]

        def out_copy(c, p):
            return pltpu.make_async_copy(
                buf_v.at[p], out_hbm.at[wbase + c], osem)

        for g in gather_copies(0, 0, gsem0):
            g.start()

        def do_group(buf, base, nrows):
            parts = []
            for r in range(nrows):
                t0 = buf[base + r, pl.ds(0, 16)]
                t1 = buf[base + r, pl.ds(16, 16)]
                t2 = buf[base + r, pl.ds(32, 16)]
                t3 = buf[base + r, pl.ds(48, 16)]
                t = (t0 * t0 + t1 * t1) + (t2 * t2 + t3 * t3)
                parts.append(jnp.where(lane == r,
                                       lax.broadcast(jnp.sum(t), (GROUP,)),
                                       0.0))
            while len(parts) > 1:
                parts = [a + b for a, b in zip(parts[::2], parts[1::2])] + (
                    [parts[-1]] if len(parts) % 2 else [])
            s = parts[0]
            s = jnp.maximum(s, EPS)
            y = _rsqrt_newton(s)
            n = s * y
            en = jnp.exp(n)
            ien = 1.0 / en
            cosh = 0.5 * (en + ien)
            sf = 0.5 * (en - ien) * y
            for r in range(nrows):
                sfr = lax.broadcast(sf[r], (GROUP,))
                rr = jnp.full((GROUP,), base + r, jnp.int32)
                vs = [buf[base + r, pl.ds(16 * q, 16)] for q in range(4)]
                for q in range(4):
                    plsc.store_scatter(buf, [rr, 1 + 16 * q + lane],
                                       sfr * vs[q])
            plsc.store_scatter(buf, [base + lane,
                                     jnp.zeros((GROUP,), jnp.int32)], cosh,
                               mask=lane < nrows)

        def chunk_body(c, _):
            p = c % 2

            @pl.when(p == 0)
            def _w0():
                for g in gather_copies(c, 0, gsem0):
                    g.wait()

            @pl.when(p == 1)
            def _w1():
                for g in gather_copies(c, 1, gsem1):
                    g.wait()

            @pl.when(c >= 1)
            def _wo():
                out_copy(c - 1, 1 - p).wait()

            @pl.when(c + 1 < bpw)
            def _g():
                @pl.when(p == 0)
                def _g1():
                    for g in gather_copies(c + 1, 1, gsem1):
                        g.start()

                @pl.when(p == 1)
                def _g0():
                    for g in gather_copies(c + 1, 0, gsem0):
                        g.start()

            buf = buf_v.at[p]

            @plsc.parallel_loop(0, n_full, step=1, unroll=4)
            def _groups(g):
                do_group(buf, g * GROUP, GROUP)
            if tail:
                do_group(buf, n_full * GROUP, tail)
            out_copy(c, p).start()
            return _

        lax.fori_loop(0, bpw, chunk_body, 0)
        out_copy(bpw - 1, (bpw - 1) % 2).wait()

    return k


def kernel(x, embed_weight):
    b, h = x.shape
    idx = x.reshape(N_WORKERS, b // N_WORKERS, 2, h // 2).astype(jnp.int32)
    tbl = jnp.pad(embed_weight, ((0, 0), (0, DP - D)))
    out = _sc_kernel(b, h)(idx, tbl)
    return lax.slice(out, (0, 0, 0), (b, h, DO))


# speedup vs baseline: 1.1591x; 1.0197x over previous
import functools

import jax
import jax.numpy as jnp
from jax import lax
from jax.experimental import pallas as pl
from jax.experimental.pallas import tpu as pltpu
from jax.experimental.pallas import tpu_sc as plsc

N_WORKERS = 32
GROUP = 16
D = 64
DO = 65
DP = 128
EPS = 1e-8


def _rsqrt_newton(s):
    i = plsc.bitcast(s, jnp.int32)
    i = jnp.int32(0x5F3759DF) - (i >> 1)
    y = plsc.bitcast(i, jnp.float32)
    for _ in range(3):
        y = y * (1.5 - 0.5 * s * y * y)
    return y


def _sc_kernel(b, h):
    bpw = b // N_WORKERS
    chunk = h
    half = chunk // 2
    n_full = chunk // GROUP
    tail = chunk - n_full * GROUP
    mesh = plsc.VectorSubcoreMesh(core_axis_name="c", subcore_axis_name="s")

    @functools.partial(
        pl.kernel,
        mesh=mesh,
        out_type=jax.ShapeDtypeStruct((b, h, DP), jnp.float32),
        scratch_types=[
            pltpu.VMEM((bpw, 2, half), jnp.int32),
            pltpu.VMEM((3, chunk, DP), jnp.float32),
            pltpu.SemaphoreType.DMA,
            pltpu.SemaphoreType.DMA,
            pltpu.SemaphoreType.DMA,
            pltpu.SemaphoreType.DMA,
        ],
        compiler_params=pltpu.CompilerParams(needs_layout_passes=False,
                                             use_tc_tiling_on_sc=False,
                                             skip_device_barrier=True),
    )
    def k(idx_hbm, table_hbm, out_hbm, idx_v, buf_v, gsem0, gsem1, gsem2,
          osem):
        wid = lax.axis_index("s") * 2 + lax.axis_index("c")
        wbase = wid * bpw
        pltpu.sync_copy(idx_hbm.at[wid], idx_v)

        lane = lax.iota(jnp.int32, GROUP)

        def gather_copies(c, p, sem):
            return [
                pltpu.make_async_copy(
                    table_hbm.at[idx_v.at[c, 0]],
                    buf_v.at[p, pl.ds(0, half)], sem),
                pltpu.make_async_copy(
                    table_hbm.at[idx_v.at[c, 1]],
                    buf_v.at[p, pl.ds(half, half)], sem),
            ]

        def out_copy(c, p):
            return pltpu.make_async_copy(
                buf_v.at[p], out_hbm.at[wbase + c], osem)

        for g in gather_copies(0, 0, gsem0):
            g.start()

        def do_group(buf, base, nrows):
            parts = []
            for r in range(nrows):
                t0 = buf[base + r, pl.ds(0, 16)]
                t1 = buf[base + r, pl.ds(16, 16)]
                t2 = buf[base + r, pl.ds(32, 16)]
                t3 = buf[base + r, pl.ds(48, 16)]
                t = (t0 * t0 + t1 * t1) + (t2 * t2 + t3 * t3)
                parts.append(jnp.where(lane == r,
                                       lax.broadcast(jnp.sum(t), (GROUP,)),
                                       0.0))
            while len(parts) > 1:
                parts = [a + b for a, b in zip(parts[::2], parts[1::2])] + (
                    [parts[-1]] if len(parts) % 2 else [])
            s = parts[0]
            s = jnp.maximum(s, EPS)
            y = _rsqrt_newton(s)
            n = s * y
            en = jnp.exp(n)
            ien = 1.0 / en
            cosh = 0.5 * (en + ien)
            sf = 0.5 * (en - ien) * y
            for r in range(nrows):
                sfr = lax.broadcast(sf[r], (GROUP,))
                rr = jnp.full((GROUP,), base + r, jnp.int32)
                vs = [buf[base + r, pl.ds(16 * q, 16)] for q in range(4)]
                for q in range(4):
                    plsc.store_scatter(buf, [rr, 1 + 16 * q + lane],
                                       sfr * vs[q])
            plsc.store_scatter(buf, [base + lane,
                                     jnp.zeros((GROUP,), jnp.int32)], cosh,
                               mask=lane < nrows)

        def chunk_body(c, _):
            p = c % 3

            @pl.when(p == 0)
            def _w0():
                for g in gather_copies(c, 0, gsem0):
                    g.wait()

            @pl.when(p == 1)
            def _w1():
                for g in gather_copies(c, 1, gsem1):
                    g.wait()

            @pl.when(p == 2)
            def _w2():
                for g in gather_copies(c, 2, gsem2):
                    g.wait()

            @pl.when(c >= 2)
            def _wo():
                out_copy(c - 2, (c - 2) % 3).wait()

            @pl.when(c + 1 < bpw)
            def _g():
                @pl.when((c + 1) % 3 == 0)
                def _g0():
                    for g in gather_copies(c + 1, 0, gsem0):
                        g.start()

                @pl.when((c + 1) % 3 == 1)
                def _g1():
                    for g in gather_copies(c + 1, 1, gsem1):
                        g.start()

                @pl.when((c + 1) % 3 == 2)
                def _g2():
                    for g in gather_copies(c + 1, 2, gsem2):
                        g.start()

            buf = buf_v.at[p]

            @plsc.parallel_loop(0, n_full, step=1, unroll=4)
            def _groups(g):
                do_group(buf, g * GROUP, GROUP)
            if tail:
                do_group(buf, n_full * GROUP, tail)
            out_copy(c, p).start()
            return _

        lax.fori_loop(0, bpw, chunk_body, 0)
        out_copy(bpw - 2, (bpw - 2) % 3).wait()
        out_copy(bpw - 1, (bpw - 1) % 3).wait()

    return k


def kernel(x, embed_weight):
    b, h = x.shape
    idx = x.reshape(N_WORKERS, b // N_WORKERS, 2, h // 2).astype(jnp.int32)
    tbl = jnp.pad(embed_weight, ((0, 0), (0, DP - D)))
    out = _sc_kernel(b, h)(idx, tbl)
    return lax.slice(out, (0, 0, 0), (b, h, DO))
